# baseline (speedup 1.0000x reference)
import functools

import jax
import jax.numpy as jnp
from jax import lax
from jax.experimental import pallas as pl
from jax.experimental.pallas import tpu as pltpu
from jax.experimental.pallas import tpu_sc as plsc

B = 16384
D = 64
NC = 2
NS = 16
NW = NC * NS
BPW = B // NW


def _sc_gather(uid0, iid0, uid1, iid1, ue0, ie0, ue1, ie1):
    mesh = plsc.VectorSubcoreMesh(core_axis_name="c", subcore_axis_name="s")

    @functools.partial(
        pl.kernel,
        mesh=mesh,
        out_type=[jax.ShapeDtypeStruct((B, D), jnp.float32) for _ in range(4)],
        scratch_types=[
            pltpu.VMEM((BPW,), jnp.int32),
            pltpu.VMEM((BPW, D), jnp.float32),
            pltpu.SemaphoreType.DMA,
        ],
        compiler_params=pltpu.CompilerParams(use_tc_tiling_on_sc=False),
    )
    def k(uid0_h, iid0_h, uid1_h, iid1_h, ue0_h, ie0_h, ue1_h, ie1_h,
          u0_o, i0_o, u1_o, i1_o, idx_v, rows_v, sem):
        wid = lax.axis_index("s") * NC + lax.axis_index("c")
        base = wid * BPW
        for ids_h, tab_h, out_h in (
            (uid0_h, ue0_h, u0_o),
            (iid0_h, ie0_h, i0_o),
            (uid1_h, ue1_h, u1_o),
            (iid1_h, ie1_h, i1_o),
        ):
            pltpu.sync_copy(ids_h.at[pl.ds(base, BPW)], idx_v)
            pltpu.async_copy(tab_h.at[idx_v], rows_v, sem).wait()
            pltpu.sync_copy(rows_v, out_h.at[pl.ds(base, BPW)])

    return k(uid0, iid0, uid1, iid1, ue0, ie0, ue1, ie1)


def _tc_body(u0_r, i0_r, u1_r, i1_r, s0_r, s1_r, o0_r, o1_r):
    p0 = jnp.dot(u0_r[...], s0_r[...], preferred_element_type=jnp.float32)
    o0_r[...] = jnp.sum(p0 * i0_r[...], axis=1, keepdims=True)
    p1 = jnp.dot(u1_r[...], s1_r[...], preferred_element_type=jnp.float32)
    o1_r[...] = jnp.sum(p1 * i1_r[...], axis=1, keepdims=True)


def _tc_dense(u0, i0, u1, i1, s_0, s_1):
    R = 2048
    nb = B // R
    row_spec = pl.BlockSpec((R, D), lambda i: (i, 0))
    s_spec = pl.BlockSpec((D, D), lambda i: (0, 0))
    out_spec = pl.BlockSpec((R, 1), lambda i: (i, 0))
    return pl.pallas_call(
        _tc_body,
        grid=(nb,),
        in_specs=[row_spec, row_spec, row_spec, row_spec, s_spec, s_spec],
        out_specs=[out_spec, out_spec],
        out_shape=[jax.ShapeDtypeStruct((B, 1), jnp.float32) for _ in range(2)],
    )(u0, i0, u1, i1, s_0, s_1)


def kernel(user_ids_0, item_ids_0, user_ids_1, item_ids_1,
           user_emb_0, user_emb_1, item_emb_0, item_emb_1,
           S0, St_0, St_1):
    u0, i0, u1, i1 = _sc_gather(user_ids_0, item_ids_0, user_ids_1, item_ids_1,
                                user_emb_0, item_emb_0, user_emb_1, item_emb_1)
    s_0 = jnp.concatenate([S0, St_0], axis=1)
    s_1 = jnp.concatenate([S0, St_1], axis=1)
    o0, o1 = _tc_dense(u0, i0, u1, i1, s_0, s_1)
    return jnp.concatenate([o0.reshape(1, B), o1.reshape(1, B)], axis=0)


# speedup vs baseline: 1.4270x; 1.4270x over previous
import functools

import jax
import jax.numpy as jnp
from jax import lax
from jax.experimental import pallas as pl
from jax.experimental.pallas import tpu as pltpu
from jax.experimental.pallas import tpu_sc as plsc

B = 16384
D = 64
NC = 2
NS = 16
NW = NC * NS
BPW = B // NW


CHUNK = 64


def _sc_gather(uid0, iid0, uid1, iid1, ue0, ie0, ue1, ie1):
    mesh = plsc.VectorSubcoreMesh(core_axis_name="c", subcore_axis_name="s")

    @functools.partial(
        pl.kernel,
        mesh=mesh,
        out_type=[jax.ShapeDtypeStruct((B, D), jnp.float32) for _ in range(4)],
        scratch_types=[
            pltpu.VMEM((BPW,), jnp.int32),
            pltpu.VMEM((BPW, D), jnp.float32),
            pltpu.SemaphoreType.DMA,
        ],
        compiler_params=pltpu.CompilerParams(
            use_tc_tiling_on_sc=True, needs_layout_passes=False),
    )
    def k(uid0_h, iid0_h, uid1_h, iid1_h, ue0_h, ie0_h, ue1_h, ie1_h,
          u0_o, i0_o, u1_o, i1_o, idx_v, rows_v, sem):
        wid = lax.axis_index("s") * NC + lax.axis_index("c")
        base = wid * BPW
        lane = lax.iota(jnp.int32, 16)
        for ids_h, tab_h, out_h in (
            (uid0_h, ue0_h, u0_o),
            (iid0_h, ie0_h, i0_o),
            (uid1_h, ue1_h, u1_o),
            (iid1_h, ie1_h, i1_o),
        ):
            pltpu.sync_copy(ids_h.at[pl.ds(base, BPW)], idx_v)

            def group_body(g):
                v = idx_v[pl.ds(g * 16, 16)]
                for j in range(16):
                    row = jnp.sum(jnp.where(lane == j, v, 0))
                    pltpu.async_copy(tab_h.at[row], rows_v.at[g * 16 + j], sem)
                for j in range(16):
                    pltpu.make_async_copy(
                        tab_h.at[0], rows_v.at[g * 16 + j], sem).wait()

            pl.loop(0, BPW // 16)(group_body)
            pltpu.sync_copy(rows_v, out_h.at[pl.ds(base, BPW)])

    return k(uid0, iid0, uid1, iid1, ue0, ie0, ue1, ie1)


def _tc_body(u0_r, i0_r, u1_r, i1_r, s0_r, s1_r, o0_r, o1_r):
    p0 = jnp.dot(u0_r[...], s0_r[...], preferred_element_type=jnp.float32)
    o0_r[...] = jnp.sum(p0 * i0_r[...], axis=1, keepdims=True)
    p1 = jnp.dot(u1_r[...], s1_r[...], preferred_element_type=jnp.float32)
    o1_r[...] = jnp.sum(p1 * i1_r[...], axis=1, keepdims=True)


def _tc_dense(u0, i0, u1, i1, s_0, s_1):
    R = 2048
    nb = B // R
    row_spec = pl.BlockSpec((R, D), lambda i: (i, 0))
    s_spec = pl.BlockSpec((D, D), lambda i: (0, 0))
    out_spec = pl.BlockSpec((R, 1), lambda i: (i, 0))
    return pl.pallas_call(
        _tc_body,
        grid=(nb,),
        in_specs=[row_spec, row_spec, row_spec, row_spec, s_spec, s_spec],
        out_specs=[out_spec, out_spec],
        out_shape=[jax.ShapeDtypeStruct((B, 1), jnp.float32) for _ in range(2)],
    )(u0, i0, u1, i1, s_0, s_1)


def kernel(user_ids_0, item_ids_0, user_ids_1, item_ids_1,
           user_emb_0, user_emb_1, item_emb_0, item_emb_1,
           S0, St_0, St_1):
    u0, i0, u1, i1 = _sc_gather(user_ids_0, item_ids_0, user_ids_1, item_ids_1,
                                user_emb_0, item_emb_0, user_emb_1, item_emb_1)
    s_0 = jnp.concatenate([S0, St_0], axis=1)
    s_1 = jnp.concatenate([S0, St_1], axis=1)
    o0, o1 = _tc_dense(u0, i0, u1, i1, s_0, s_1)
    return jnp.concatenate([o0.reshape(1, B), o1.reshape(1, B)], axis=0)
